# baseline (speedup 1.0000x reference)
import functools

import jax
import jax.numpy as jnp
from jax import lax
from jax.experimental import pallas as pl
from jax.experimental.pallas import tpu as pltpu
from jax.experimental.pallas import tpu_sc as plsc

_D = 96
_L = 16
_DV = _D // _L
_ROWS = 16 * 1024
_NC = 2
_NS = 16
_NW = _NC * _NS
_RPW = _ROWS // _NW

_mesh = plsc.VectorSubcoreMesh(core_axis_name="c", subcore_axis_name="s")


@functools.partial(
    pl.kernel,
    mesh=_mesh,
    out_type=jax.ShapeDtypeStruct((_ROWS, _D), jnp.float32),
    scratch_types=[
        pltpu.VMEM((_RPW, _D), jnp.float32),
        pltpu.VMEM((8, _D), jnp.float32),
    ],
    compiler_params=pltpu.CompilerParams(use_tc_tiling_on_sc=True),
)
def _encode(patch_hbm, pos_hbm, out_hbm, buf, posv):
    wid = lax.axis_index("s") * _NC + lax.axis_index("c")
    base = wid * _RPW
    pltpu.sync_copy(pos_hbm.at[pl.ds(0, 8)], posv)
    pltpu.sync_copy(patch_hbm.at[pl.ds(base, _RPW)], buf)
    pv = [posv[0, pl.ds(j * _L, _L)] for j in range(_DV)]

    def row(r, carry):
        for j in range(_DV):
            buf[r, pl.ds(j * _L, _L)] += pv[j]
        return carry

    lax.fori_loop(0, _RPW, row, 0)
    pltpu.sync_copy(buf, out_hbm.at[pl.ds(base, _RPW)])


def kernel(patch, pos_table):
    rows = patch.reshape(-1, _D)
    out = _encode(rows, pos_table)
    return out.reshape(patch.shape)


# speedup vs baseline: 1.6835x; 1.6835x over previous
import functools

import jax
import jax.numpy as jnp
from jax import lax
from jax.experimental import pallas as pl
from jax.experimental.pallas import tpu as pltpu
from jax.experimental.pallas import tpu_sc as plsc

_B = 16
_P = 1024
_D = 96
_L = 16
_TROWS = _B * _D
_NC = 2
_NS = 16
_NW = _NC * _NS
_RPW = _TROWS // _NW

_mesh = plsc.VectorSubcoreMesh(core_axis_name="c", subcore_axis_name="s")


@functools.partial(
    pl.kernel,
    mesh=_mesh,
    out_type=jax.ShapeDtypeStruct((_TROWS, _P), jnp.float32),
    scratch_types=[
        pltpu.VMEM((_RPW, _P), jnp.float32),
        pltpu.VMEM((8, _D), jnp.float32),
        pltpu.VMEM((_RPW, _L), jnp.float32),
    ],
    compiler_params=pltpu.CompilerParams(use_tc_tiling_on_sc=True),
)
def _encode(patch_hbm, pos_hbm, out_hbm, buf, posv, splats):
    wid = lax.axis_index("s") * _NC + lax.axis_index("c")
    base = wid * _RPW
    dbase = (wid % 2) * _RPW
    pltpu.sync_copy(pos_hbm.at[pl.ds(0, 8)], posv)
    pltpu.sync_copy(patch_hbm.at[pl.ds(base, _RPW)], buf)

    for g in range(_RPW // _L):
        vg = posv[0, pl.ds(dbase + g * _L, _L)]
        for k in range(_L):
            splats[g * _L + k, :] = lax.broadcast(vg[k], (_L,))

    def row(r, carry):
        pv = splats[r, :]
        for j in range(_P // _L):
            buf[r, pl.ds(j * _L, _L)] += pv
        return carry

    lax.fori_loop(0, _RPW, row, 0)
    pltpu.sync_copy(buf, out_hbm.at[pl.ds(base, _RPW)])


def kernel(patch, pos_table):
    pt = patch.transpose(0, 2, 1).reshape(_TROWS, _P)
    out = _encode(pt, pos_table)
    return out.reshape(_B, _D, _P).transpose(0, 2, 1)


# speedup vs baseline: 1.6988x; 1.0091x over previous
import functools

import jax
import jax.numpy as jnp
from jax import lax
from jax.experimental import pallas as pl
from jax.experimental.pallas import tpu as pltpu
from jax.experimental.pallas import tpu_sc as plsc

_B = 16
_P = 1024
_D = 96
_L = 16
_TROWS = _B * _D
_NC = 2
_NS = 16
_NW = _NC * _NS
_RPW = _TROWS // _NW
_CH = 8
_NCH = _RPW // _CH

_mesh = plsc.VectorSubcoreMesh(core_axis_name="c", subcore_axis_name="s")


@functools.partial(
    pl.kernel,
    mesh=_mesh,
    out_type=jax.ShapeDtypeStruct((_TROWS, _P), jnp.float32),
    scratch_types=[
        pltpu.VMEM((_CH, _P), jnp.float32),
        pltpu.VMEM((_CH, _P), jnp.float32),
        pltpu.VMEM((_D,), jnp.float32),
        pltpu.VMEM((_RPW, _L), jnp.float32),
        pltpu.SemaphoreType.DMA,
        pltpu.SemaphoreType.DMA,
        pltpu.SemaphoreType.DMA,
        pltpu.SemaphoreType.DMA,
    ],
    compiler_params=pltpu.CompilerParams(use_tc_tiling_on_sc=True),
)
def _encode(patch_hbm, pos_hbm, out_hbm, buf0, buf1, posv, splats,
            lsem0, lsem1, ssem0, ssem1):
    wid = lax.axis_index("s") * _NC + lax.axis_index("c")
    base = wid * _RPW
    dbase = (wid % 2) * _RPW
    bufs = (buf0, buf1)
    lsems = (lsem0, lsem1)
    ssems = (ssem0, ssem1)

    loads = [None] * _NCH
    stores = [None] * _NCH
    loads[0] = pltpu.async_copy(patch_hbm.at[pl.ds(base, _CH)], bufs[0], lsems[0])

    pltpu.sync_copy(pos_hbm, posv)
    for g in range(_RPW // _L):
        vg = posv[pl.ds(dbase + g * _L, _L)]
        for k in range(_L):
            splats[g * _L + k, :] = lax.broadcast(vg[k], (_L,))

    for c in range(_NCH):
        b = c % 2
        if c + 1 < _NCH:
            if c >= 1:
                stores[c - 1].wait()
            loads[c + 1] = pltpu.async_copy(
                patch_hbm.at[pl.ds(base + (c + 1) * _CH, _CH)],
                bufs[(c + 1) % 2], lsems[(c + 1) % 2])
        loads[c].wait()
        buf = bufs[b]

        def row(r, carry, buf=buf, c=c):
            pv = splats[c * _CH + r, :]
            for j in range(_P // _L):
                buf[r, pl.ds(j * _L, _L)] += pv
            return carry

        lax.fori_loop(0, _CH, row, 0)
        stores[c] = pltpu.async_copy(
            buf, out_hbm.at[pl.ds(base + c * _CH, _CH)], ssems[b])

    stores[_NCH - 2].wait()
    stores[_NCH - 1].wait()


def kernel(patch, pos_table):
    pt = patch.transpose(0, 2, 1).reshape(_TROWS, _P)
    out = _encode(pt, pos_table[0])
    return out.reshape(_B, _D, _P).transpose(0, 2, 1)
